# baseline (speedup 1.0000x reference)
import functools

import jax
import jax.numpy as jnp
from jax import lax
from jax.experimental import pallas as pl
from jax.experimental.pallas import tpu as pltpu
from jax.experimental.pallas import tpu_sc as plsc

VOCAB = 100000
H = 768
POS = 4096
B = 4
S = 4096

L = 16
NB = H // L
NW = 32
ROWS = B * S
RPW = ROWS // NW
CH = 32
NCH = RPW // CH
SCALE = float(H) ** 0.5
EPS = 1e-5


_GDN = lax.GatherDimensionNumbers(
    offset_dims=(), collapsed_slice_dims=(0,), start_index_map=(0,))


def _perm(v, idx):
    return lax.gather(v, idx.reshape(L, 1), _GDN, slice_sizes=(1,),
                      mode=lax.GatherScatterMode.PROMISE_IN_BOUNDS)


def _lane_sum(v):
    lanes = lax.iota(jnp.int32, L)
    for sh in (1, 2, 4, 8):
        v = v + _perm(v, (lanes + sh) & (L - 1))
    return v


def _rsqrt_vec(v):
    i = plsc.bitcast(v, jnp.int32)
    i = jnp.int32(0x5F3759DF) - lax.shift_right_logical(i, 1)
    y = plsc.bitcast(i, jnp.float32)
    for _ in range(3):
        y = y * (1.5 - 0.5 * v * y * y)
    return y


_mesh = plsc.VectorSubcoreMesh(core_axis_name="c", subcore_axis_name="s")


@functools.partial(
    pl.kernel,
    out_type=jax.ShapeDtypeStruct((ROWS, H), jnp.float32),
    mesh=_mesh,
    compiler_params=pltpu.CompilerParams(needs_layout_passes=False),
    scratch_types=[
        pltpu.VMEM((CH,), jnp.int32),
        pltpu.VMEM((CH,), jnp.int32),
        pltpu.VMEM((CH, H), jnp.float32),
        pltpu.VMEM((CH, H), jnp.float32),
        pltpu.VMEM((2, H), jnp.float32),
        pltpu.VMEM((H,), jnp.float32),
        pltpu.VMEM((H,), jnp.float32),
        pltpu.SemaphoreType.DMA,
    ],
)
def _emb_ln_kernel(ids_hbm, seg_hbm, ww_hbm, wseg_hbm, gamma_hbm, beta_hbm,
                   pos_hbm, out_hbm, idx_v, segv, rows_v, pos_v, wseg_v,
                   gamma_v, beta_v, sem):
    wid = lax.axis_index("s") * 2 + lax.axis_index("c")
    base = wid * RPW
    sbase = lax.rem(base, S)

    pltpu.sync_copy(wseg_hbm, wseg_v)
    pltpu.sync_copy(gamma_hbm, gamma_v)
    pltpu.sync_copy(beta_hbm, beta_v)

    def chunk_body(c, carry):
        off = base + c * CH
        soff = sbase + c * CH
        pltpu.sync_copy(ids_hbm.at[pl.ds(off, CH)], idx_v)
        pltpu.sync_copy(seg_hbm.at[pl.ds(off, CH)], segv)
        pltpu.sync_copy(pos_hbm.at[pl.ds(soff, CH)], pos_v)
        pltpu.async_copy(ww_hbm.at[idx_v], rows_v, sem).wait()

        for g in range(CH // L):
            sgrp = segv[pl.ds(g * L, L)]

            def row_body(i, carry2, sgrp=sgrp, g=g):
                row = g * L + i
                sidf = _perm(sgrp, jnp.full((L,), i, jnp.int32)
                             ).astype(jnp.float32)
                acc_s = jnp.zeros((L,), jnp.float32)
                acc_q = jnp.zeros((L,), jnp.float32)
                for j in range(NB):
                    d = pl.ds(j * L, L)
                    x = (rows_v[row, d] * SCALE + pos_v[row, d]
                         + wseg_v[0, d] + sidf * (wseg_v[1, d] - wseg_v[0, d]))
                    rows_v[row, d] = x
                    acc_s = acc_s + x
                    acc_q = acc_q + x * x
                mub = _lane_sum(acc_s) * (1.0 / H)
                varb = _lane_sum(acc_q) * (1.0 / H) - mub * mub
                rib = _rsqrt_vec(varb + EPS)
                for j in range(NB):
                    d = pl.ds(j * L, L)
                    rows_v[row, d] = ((rows_v[row, d] - mub) * rib * gamma_v[d]
                                      + beta_v[d])
                return carry2

            lax.fori_loop(0, L, row_body, 0)
        pltpu.sync_copy(rows_v, out_hbm.at[pl.ds(off, CH)])
        return carry

    lax.fori_loop(0, NCH, chunk_body, 0)


def kernel(input_ids, segment_ids, W_word, W_seg, gamma, beta, pos_enc):
    ids = input_ids.reshape(ROWS).astype(jnp.int32)
    seg = segment_ids.reshape(ROWS).astype(jnp.int32)
    out = _emb_ln_kernel(ids, seg, W_word, W_seg, gamma, beta, pos_enc)
    return out.reshape(B, S, H)


# speedup vs baseline: 4.0299x; 4.0299x over previous
import functools

import jax
import jax.numpy as jnp
from jax import lax
from jax.experimental import pallas as pl
from jax.experimental.pallas import tpu as pltpu
from jax.experimental.pallas import tpu_sc as plsc

VOCAB = 100000
H = 768
POS = 4096
B = 4
S = 4096

NW = 32
ROWS = B * S
RPW = ROWS // NW
CH = 64
NCH = RPW // CH
SCALE = float(H) ** 0.5
EPS = 1e-5

TR = 512
NTB = ROWS // TR
SB = S // TR

_mesh = plsc.VectorSubcoreMesh(core_axis_name="c", subcore_axis_name="s")


@functools.partial(
    pl.kernel,
    out_type=jax.ShapeDtypeStruct((ROWS, H), jnp.float32),
    mesh=_mesh,
    compiler_params=pltpu.CompilerParams(needs_layout_passes=False),
    scratch_types=[
        pltpu.VMEM((2, CH), jnp.int32),
        pltpu.VMEM((2, CH, H), jnp.float32),
        pltpu.SemaphoreType.DMA,
        pltpu.SemaphoreType.DMA,
    ],
)
def _gather_kernel(ids_hbm, ww_hbm, out_hbm, idx_v, rows_v, sem0, sem1):
    wid = lax.axis_index("s") * 2 + lax.axis_index("c")
    base = wid * RPW
    sems = (sem0, sem1)
    copies = [None, None]
    for p in range(2):
        pltpu.sync_copy(ids_hbm.at[pl.ds(base + p * CH, CH)], idx_v.at[p])
        copies[p] = pltpu.async_copy(ww_hbm.at[idx_v.at[p]], rows_v.at[p],
                                     sems[p])
    for c in range(NCH):
        b = c & 1
        copies[b].wait()
        if c + 2 < NCH:
            pltpu.sync_copy(ids_hbm.at[pl.ds(base + (c + 2) * CH, CH)],
                            idx_v.at[b])
        pltpu.sync_copy(rows_v.at[b], out_hbm.at[pl.ds(base + c * CH, CH)])
        if c + 2 < NCH:
            copies[b] = pltpu.async_copy(ww_hbm.at[idx_v.at[b]], rows_v.at[b],
                                         sems[b])


def _ln_body(g_ref, p_ref, s_ref, ws_ref, ga_ref, be_ref, o_ref):
    x = g_ref[...] * SCALE + p_ref[...]
    sidf = s_ref[0, 0, :].astype(jnp.float32)[:, None]
    x = x + ws_ref[0:1, :] + sidf * (ws_ref[1:2, :] - ws_ref[0:1, :])
    mu = jnp.mean(x, axis=-1, keepdims=True)
    var = jnp.mean(x * x, axis=-1, keepdims=True) - mu * mu
    o_ref[...] = (x - mu) * lax.rsqrt(var + EPS) * ga_ref[...] + be_ref[...]


_ln_call = pl.pallas_call(
    _ln_body,
    grid=(NTB,),
    in_specs=[
        pl.BlockSpec((TR, H), lambda i: (i, 0)),
        pl.BlockSpec((TR, H), lambda i: (i % SB, 0)),
        pl.BlockSpec((1, 1, TR), lambda i: (i, 0, 0)),
        pl.BlockSpec((2, H), lambda i: (0, 0)),
        pl.BlockSpec((1, H), lambda i: (0, 0)),
        pl.BlockSpec((1, H), lambda i: (0, 0)),
    ],
    out_specs=pl.BlockSpec((TR, H), lambda i: (i, 0)),
    out_shape=jax.ShapeDtypeStruct((ROWS, H), jnp.float32),
)


def kernel(input_ids, segment_ids, W_word, W_seg, gamma, beta, pos_enc):
    ids = input_ids.reshape(ROWS).astype(jnp.int32)
    seg3 = segment_ids.reshape(NTB, 1, TR).astype(jnp.int32)
    gathered = _gather_kernel(ids, W_word)
    out = _ln_call(gathered, pos_enc, seg3, W_seg,
                   gamma.reshape(1, H), beta.reshape(1, H))
    return out.reshape(B, S, H)
